# baseline (speedup 1.0000x reference)
import jax
import jax.numpy as jnp
from jax.experimental import pallas as pl
from jax.experimental.pallas import tpu as pltpu

_N = 100000
_BLK = 2000


def _mlp_block(mask_ref, x_ref, wd4_ref, bd4_ref, wg0_ref, bg0_ref,
               wg1_ref, bg1_ref, wh_ref, bh_ref,
               dxyz_ref, drot_ref, dscale_ref):
    x = x_ref[...]
    h = jax.nn.relu(jnp.dot(x, wd4_ref[...],
                            preferred_element_type=jnp.float32) + bd4_ref[...])
    h = jax.nn.relu(jnp.dot(h, wg0_ref[...],
                            preferred_element_type=jnp.float32) + bg0_ref[...])
    h = jax.nn.relu(jnp.dot(h, wg1_ref[...],
                            preferred_element_type=jnp.float32) + bg1_ref[...])
    y = jnp.dot(h, wh_ref[...], preferred_element_type=jnp.float32) + bh_ref[...]
    y = y * mask_ref[...]
    dxyz_ref[...] = y[:, 0:3]
    dscale_ref[...] = y[:, 3:6]
    drot_ref[...] = y[:, 6:10]


def kernel(mask, t, spatial_dxyz, d4_h, W_d4, b_d4, W_g0, b_g0, W_g1, b_g1,
           W_warp, b_warp, W_scale, b_scale, W_rot, b_rot):
    n = mask.shape[0]
    mask_f = mask.astype(jnp.float32)[:, None]
    w_heads = jnp.concatenate([W_warp, W_scale, W_rot], axis=1)
    b_heads = jnp.concatenate([b_warp, b_scale, b_rot])[None, :]

    grid = (n // _BLK,)
    row_spec = lambda width: pl.BlockSpec((_BLK, width), lambda i: (i, 0))
    full_spec = lambda a: pl.BlockSpec(a.shape, lambda i: (0,) * a.ndim)

    d_xyz, d_rotation, d_scaling = pl.pallas_call(
        _mlp_block,
        grid=grid,
        in_specs=[
            row_spec(1),
            row_spec(256),
            full_spec(W_d4), full_spec(b_d4[None, :]),
            full_spec(W_g0), full_spec(b_g0[None, :]),
            full_spec(W_g1), full_spec(b_g1[None, :]),
            full_spec(w_heads), full_spec(b_heads),
        ],
        out_specs=[row_spec(3), row_spec(4), row_spec(3)],
        out_shape=[
            jax.ShapeDtypeStruct((n, 3), jnp.float32),
            jax.ShapeDtypeStruct((n, 4), jnp.float32),
            jax.ShapeDtypeStruct((n, 3), jnp.float32),
        ],
        compiler_params=pltpu.CompilerParams(
            dimension_semantics=("parallel",)),
    )(mask_f, d4_h, W_d4, b_d4[None, :], W_g0, b_g0[None, :],
      W_g1, b_g1[None, :], w_heads, b_heads)

    d_opacity = jnp.zeros((n, 1), dtype=jnp.float32)
    d_shs = jnp.zeros((n, 16, 3), dtype=jnp.float32)
    return (d_xyz, d_rotation, d_scaling, d_opacity, d_shs)


# speedup vs baseline: 1.0007x; 1.0007x over previous
import jax
import jax.numpy as jnp
from jax.experimental import pallas as pl
from jax.experimental.pallas import tpu as pltpu

_N = 100000
_BLK = 2000


def _dot(a, b):
    return jnp.dot(a.astype(jnp.bfloat16), b.astype(jnp.bfloat16),
                   preferred_element_type=jnp.float32)


def _mlp_block(mask_ref, x_ref, wd4_ref, bd4_ref, wg0_ref, bg0_ref,
               wg1_ref, bg1_ref, wh_ref, bh_ref,
               dxyz_ref, drot_ref, dscale_ref):
    x = x_ref[...]
    h = jax.nn.relu(_dot(x, wd4_ref[...]) + bd4_ref[...])
    h = jax.nn.relu(_dot(h, wg0_ref[...]) + bg0_ref[...])
    h = jax.nn.relu(_dot(h, wg1_ref[...]) + bg1_ref[...])
    y = _dot(h, wh_ref[...]) + bh_ref[...]
    y = y * mask_ref[...]
    dxyz_ref[...] = y[:, 0:3]
    dscale_ref[...] = y[:, 3:6]
    drot_ref[...] = y[:, 6:10]


def kernel(mask, t, spatial_dxyz, d4_h, W_d4, b_d4, W_g0, b_g0, W_g1, b_g1,
           W_warp, b_warp, W_scale, b_scale, W_rot, b_rot):
    n = mask.shape[0]
    mask_f = mask.astype(jnp.float32)[:, None]
    w_heads = jnp.concatenate([W_warp, W_scale, W_rot], axis=1)
    b_heads = jnp.concatenate([b_warp, b_scale, b_rot])[None, :]

    grid = (n // _BLK,)
    row_spec = lambda width: pl.BlockSpec((_BLK, width), lambda i: (i, 0))
    full_spec = lambda a: pl.BlockSpec(a.shape, lambda i: (0,) * a.ndim)

    d_xyz, d_rotation, d_scaling = pl.pallas_call(
        _mlp_block,
        grid=grid,
        in_specs=[
            row_spec(1),
            row_spec(256),
            full_spec(W_d4), full_spec(b_d4[None, :]),
            full_spec(W_g0), full_spec(b_g0[None, :]),
            full_spec(W_g1), full_spec(b_g1[None, :]),
            full_spec(w_heads), full_spec(b_heads),
        ],
        out_specs=[row_spec(3), row_spec(4), row_spec(3)],
        out_shape=[
            jax.ShapeDtypeStruct((n, 3), jnp.float32),
            jax.ShapeDtypeStruct((n, 4), jnp.float32),
            jax.ShapeDtypeStruct((n, 3), jnp.float32),
        ],
        compiler_params=pltpu.CompilerParams(
            dimension_semantics=("parallel",)),
    )(mask_f, d4_h, W_d4, b_d4[None, :], W_g0, b_g0[None, :],
      W_g1, b_g1[None, :], w_heads, b_heads)

    d_opacity = jnp.zeros((n, 1), dtype=jnp.float32)
    d_shs = jnp.zeros((n, 16, 3), dtype=jnp.float32)
    return (d_xyz, d_rotation, d_scaling, d_opacity, d_shs)


# speedup vs baseline: 1.0802x; 1.0794x over previous
import jax
import jax.numpy as jnp
from jax.experimental import pallas as pl
from jax.experimental.pallas import tpu as pltpu

_N = 100000
_BLK = 5000


def _dot(a, b):
    return jnp.dot(a.astype(jnp.bfloat16), b.astype(jnp.bfloat16),
                   preferred_element_type=jnp.float32)


def _mlp_block(mask_ref, x_ref, wd4_ref, bd4_ref, wg0_ref, bg0_ref,
               wg1_ref, bg1_ref, wh_ref, bh_ref,
               dxyz_ref, drot_ref, dscale_ref):
    x = x_ref[...]
    h = jax.nn.relu(_dot(x, wd4_ref[...]) + bd4_ref[...])
    h = jax.nn.relu(_dot(h, wg0_ref[...]) + bg0_ref[...])
    h = jax.nn.relu(_dot(h, wg1_ref[...]) + bg1_ref[...])
    y = _dot(h, wh_ref[...]) + bh_ref[...]
    y = y * mask_ref[...]
    dxyz_ref[...] = y[:, 0:3]
    dscale_ref[...] = y[:, 3:6]
    drot_ref[...] = y[:, 6:10]


def kernel(mask, t, spatial_dxyz, d4_h, W_d4, b_d4, W_g0, b_g0, W_g1, b_g1,
           W_warp, b_warp, W_scale, b_scale, W_rot, b_rot):
    n = mask.shape[0]
    mask_f = mask.astype(jnp.float32)[:, None]
    w_heads = jnp.concatenate([W_warp, W_scale, W_rot], axis=1)
    b_heads = jnp.concatenate([b_warp, b_scale, b_rot])[None, :]

    grid = (n // _BLK,)
    row_spec = lambda width: pl.BlockSpec((_BLK, width), lambda i: (i, 0))
    full_spec = lambda a: pl.BlockSpec(a.shape, lambda i: (0,) * a.ndim)

    d_xyz, d_rotation, d_scaling = pl.pallas_call(
        _mlp_block,
        grid=grid,
        in_specs=[
            row_spec(1),
            row_spec(256),
            full_spec(W_d4), full_spec(b_d4[None, :]),
            full_spec(W_g0), full_spec(b_g0[None, :]),
            full_spec(W_g1), full_spec(b_g1[None, :]),
            full_spec(w_heads), full_spec(b_heads),
        ],
        out_specs=[row_spec(3), row_spec(4), row_spec(3)],
        out_shape=[
            jax.ShapeDtypeStruct((n, 3), jnp.float32),
            jax.ShapeDtypeStruct((n, 4), jnp.float32),
            jax.ShapeDtypeStruct((n, 3), jnp.float32),
        ],
        compiler_params=pltpu.CompilerParams(
            dimension_semantics=("parallel",)),
    )(mask_f, d4_h, W_d4, b_d4[None, :], W_g0, b_g0[None, :],
      W_g1, b_g1[None, :], w_heads, b_heads)

    d_opacity = jnp.zeros((n, 1), dtype=jnp.float32)
    d_shs = jnp.zeros((n, 16, 3), dtype=jnp.float32)
    return (d_xyz, d_rotation, d_scaling, d_opacity, d_shs)
